# baseline (speedup 1.0000x reference)
import functools

import jax
import jax.numpy as jnp
from jax import lax
from jax.experimental import pallas as pl
from jax.experimental.pallas import tpu as pltpu
from jax.experimental.pallas import tpu_sc as plsc

E = 1_600_000
R = 19
NT = 16
NW = 32
NBLK = E // 128
CBLK = 14
C = CBLK * 128
NCH = 28
G = C // 16

_LN2 = 0.6931471805599453
_BOHR = 1.8897259886
_RS = 6.0
_WINV = 5.0


def _lnpoly(z):
    z2 = z * z
    return z * (2.0 + z2 * (0.6666666666 + z2 * (0.4 + z2 * 0.2857142857)))


def _ln16(x):
    bits = lax.bitcast_convert_type(x, jnp.int32)
    e = (bits >> 23) - 127
    m = lax.bitcast_convert_type((bits & 0x007FFFFF) | 0x3F800000, jnp.float32)
    big = m > 1.4142135623730951
    m = jnp.where(big, m * 0.5, m)
    ef = e.astype(jnp.float32) + jnp.where(big, 1.0, 0.0)
    z = (m - 1.0) / (m + 1.0)
    return ef * _LN2 + _lnpoly(z)


def _body(et_hbm, en_hbm, rij_hbm, tab_hbm, r0p_hbm, out_hbm,
          tab_v, lr0_v,
          etA, en0A, en1A, rijA, outA,
          etB, en0B, en1B, rijB, outB,
          sinA, sinB, soutA, soutB):
    cid = lax.axis_index("c")
    sid = lax.axis_index("s")
    wid = sid * 2 + cid

    pltpu.sync_copy(tab_hbm, tab_v)
    pltpu.sync_copy(r0p_hbm, lr0_v)

    lr0_v[...] = _ln16(lr0_v[...])

    pkc = []
    for k in range(R):
        a1 = tab_v[pl.ds((2 * k) * 16, 16)]
        a2 = tab_v[pl.ds((2 * k + 1) * 16, 16)]
        hi = (lax.bitcast_convert_type(a1, jnp.int32) + 0x8000) & jnp.int32(-65536)
        lo = ((lax.bitcast_convert_type(jnp.abs(a2), jnp.int32) + 0x8000) >> 16) & 0xFFFF
        pkc.append(hi | lo)

    nb0 = (wid * (NBLK // 4)) >> 3
    nb1 = ((wid + 1) * (NBLK // 4)) >> 3
    gsm = lax.GatherScatterMode.PROMISE_IN_BOUNDS

    def cb_of(c):
        return jnp.minimum(nb0 + c * CBLK, nb1 - CBLK) * 128

    def issue_in(c, et_v, en0_v, en1_v, rij_v, sem):
        cb = cb_of(c)
        pltpu.async_copy(et_hbm.at[pl.ds(cb, C)], et_v, sem)
        pltpu.async_copy(en_hbm.at[pl.ds(cb, C)], en0_v, sem)
        pltpu.async_copy(en_hbm.at[pl.ds(E + cb, C)], en1_v, sem)
        pltpu.async_copy(rij_hbm.at[pl.ds(cb, C)], rij_v, sem)

    def wait_in(et_v, en0_v, en1_v, rij_v, sem):
        pltpu.make_async_copy(et_hbm.at[pl.ds(0, C)], et_v, sem).wait()
        pltpu.make_async_copy(en_hbm.at[pl.ds(0, C)], en0_v, sem).wait()
        pltpu.make_async_copy(en_hbm.at[pl.ds(0, C)], en1_v, sem).wait()
        pltpu.make_async_copy(rij_hbm.at[pl.ds(0, C)], rij_v, sem).wait()

    def issue_out(c, out_v, sem):
        pltpu.async_copy(out_v, out_hbm.at[:, pl.ds(cb_of(c), C)], sem)

    def wait_out(out_v, sem):
        pltpu.make_async_copy(out_v, out_hbm.at[:, pl.ds(0, C)], sem).wait()

    def compute(et_v, en0_v, en1_v, rij_v, out_v):
        def group(g, carry2):
            o = pl.multiple_of(g * 16, 16)
            et = et_v[pl.ds(o, 16)]
            n0 = en0_v[pl.ds(o, 16)]
            n1 = en1_v[pl.ds(o, 16)]
            rij = rij_v[pl.ds(o, 16)]
            lr0 = plsc.load_gather(lr0_v, [n0 * 4 + n1])
            lrr = lr0 - _ln16(rij)
            t = jnp.exp((rij - _RS) * _WINV)
            base = lrr - _lnpoly(t / (2.0 + t))
            for k in range(R):
                pk = pkc[k].at[et].get(mode=gsm)
                a1 = lax.bitcast_convert_type(pk & jnp.int32(-65536), jnp.float32)
                e2m1 = lax.bitcast_convert_type(pk << 16, jnp.float32)
                y = a1 * jnp.exp(e2m1 * lrr + base)
                out_v[k, pl.ds(o, 16)] = y
            return carry2

        lax.fori_loop(0, G, group, 0)

    bufA = (etA, en0A, en1A, rijA)
    bufB = (etB, en0B, en1B, rijB)

    issue_in(0, *bufA, sinA)

    def big(c2, carry):
        c0 = c2 * 2
        c1 = c0 + 1
        wait_in(*bufA, sinA)
        issue_in(c1, *bufB, sinB)

        @pl.when(c2 > 0)
        def _():
            wait_out(outA, soutA)

        compute(*bufA, outA)
        issue_out(c0, outA, soutA)

        wait_in(*bufB, sinB)

        @pl.when(c2 < NCH // 2 - 1)
        def _():
            issue_in(c0 + 2, *bufA, sinA)

        @pl.when(c2 > 0)
        def _():
            wait_out(outB, soutB)

        compute(*bufB, outB)
        issue_out(c1, outB, soutB)
        return carry

    lax.fori_loop(0, NCH // 2, big, 0)
    wait_out(outA, soutA)
    wait_out(outB, soutB)


_sk = functools.partial(
    pl.kernel,
    out_type=jax.ShapeDtypeStruct((R, E), jnp.float32),
    mesh=plsc.VectorSubcoreMesh(core_axis_name="c", subcore_axis_name="s"),
    compiler_params=pltpu.CompilerParams(
        needs_layout_passes=False, use_tc_tiling_on_sc=True),
    scratch_types=[
        pltpu.VMEM((2 * R * 16,), jnp.float32),
        pltpu.VMEM((16,), jnp.float32),
        pltpu.VMEM((C,), jnp.int32),
        pltpu.VMEM((C,), jnp.int32),
        pltpu.VMEM((C,), jnp.int32),
        pltpu.VMEM((C,), jnp.float32),
        pltpu.VMEM((R, C), jnp.float32),
        pltpu.VMEM((C,), jnp.int32),
        pltpu.VMEM((C,), jnp.int32),
        pltpu.VMEM((C,), jnp.int32),
        pltpu.VMEM((C,), jnp.float32),
        pltpu.VMEM((R, C), jnp.float32),
        pltpu.SemaphoreType.DMA,
        pltpu.SemaphoreType.DMA,
        pltpu.SemaphoreType.DMA,
        pltpu.SemaphoreType.DMA,
    ],
)(_body)


def kernel(edge_type, edge_number, rij, hopping_param, atomic_radius):
    tabT = jnp.transpose(hopping_param, (1, 2, 0)).reshape(2 * R * NT)
    r0p = ((atomic_radius[:, None] + atomic_radius[None, :]) / _BOHR).reshape(NT)
    en_flat = edge_number.reshape(2 * E)
    out = _sk(edge_type, en_flat, rij, tabT, r0p)
    return out.T


# speedup vs baseline: 1.1407x; 1.1407x over previous
import functools

import jax
import jax.numpy as jnp
from jax import lax
from jax.experimental import pallas as pl
from jax.experimental.pallas import tpu as pltpu
from jax.experimental.pallas import tpu_sc as plsc

E = 1_600_000
R = 19
NT = 16
NW = 32
NBLK = E // 128
CBLK = 14
C = CBLK * 128
NCH = 28
G = C // 16

_LN2 = 0.6931471805599453
_BOHR = 1.8897259886
_RS = 6.0
_WINV = 5.0


def _lnpoly(z):
    z2 = z * z
    return z * (2.0 + z2 * (0.6666666666 + z2 * (0.4 + z2 * 0.2857142857)))


def _ln16(x):
    bits = lax.bitcast_convert_type(x, jnp.int32)
    e = (bits >> 23) - 127
    m = lax.bitcast_convert_type((bits & 0x007FFFFF) | 0x3F800000, jnp.float32)
    big = m > 1.4142135623730951
    m = jnp.where(big, m * 0.5, m)
    ef = e.astype(jnp.float32) + jnp.where(big, 1.0, 0.0)
    z = (m - 1.0) / (m + 1.0)
    return ef * _LN2 + _lnpoly(z)


def _body(et_hbm, en_hbm, rij_hbm, tab_hbm, r0p_hbm, out_hbm,
          tab_v, lr0_v,
          etA, en0A, en1A, rijA, outA,
          etB, en0B, en1B, rijB, outB,
          sinA, sinB, soutA, soutB):
    cid = lax.axis_index("c")
    sid = lax.axis_index("s")
    wid = sid * 2 + cid

    pltpu.sync_copy(tab_hbm, tab_v)
    pltpu.sync_copy(r0p_hbm, lr0_v)

    lr0_v[...] = _ln16(lr0_v[...])

    pkc = []
    for k in range(R):
        a1 = tab_v[pl.ds((2 * k) * 16, 16)]
        a2 = tab_v[pl.ds((2 * k + 1) * 16, 16)]
        hi = (lax.bitcast_convert_type(a1, jnp.int32) + 0x8000) & jnp.int32(-65536)
        lo = ((lax.bitcast_convert_type(jnp.abs(a2), jnp.int32) + 0x8000) >> 16) & 0xFFFF
        pkc.append(hi | lo)

    nb0 = (wid * (NBLK // 4)) >> 3
    nb1 = ((wid + 1) * (NBLK // 4)) >> 3
    gsm = lax.GatherScatterMode.PROMISE_IN_BOUNDS

    def cb_of(c):
        return jnp.minimum(nb0 + c * CBLK, nb1 - CBLK) * 128

    def issue_in(c, et_v, en0_v, en1_v, rij_v, sem):
        cb = cb_of(c)
        pltpu.async_copy(et_hbm.at[pl.ds(cb, C)], et_v, sem)
        pltpu.async_copy(en_hbm.at[pl.ds(cb, C)], en0_v, sem)
        pltpu.async_copy(en_hbm.at[pl.ds(E + cb, C)], en1_v, sem)
        pltpu.async_copy(rij_hbm.at[pl.ds(cb, C)], rij_v, sem)

    def wait_in(et_v, en0_v, en1_v, rij_v, sem):
        pltpu.make_async_copy(et_hbm.at[pl.ds(0, C)], et_v, sem).wait()
        pltpu.make_async_copy(en_hbm.at[pl.ds(0, C)], en0_v, sem).wait()
        pltpu.make_async_copy(en_hbm.at[pl.ds(0, C)], en1_v, sem).wait()
        pltpu.make_async_copy(rij_hbm.at[pl.ds(0, C)], rij_v, sem).wait()

    def issue_out(c, out_v, sem):
        pltpu.async_copy(out_v, out_hbm.at[:, pl.ds(cb_of(c), C)], sem)

    def wait_out(out_v, sem):
        pltpu.make_async_copy(out_v, out_hbm.at[:, pl.ds(0, C)], sem).wait()

    def compute(et_v, en0_v, en1_v, rij_v, out_v):
        def group(g):
            o = pl.multiple_of(g * 16, 16)
            et = et_v[pl.ds(o, 16)]
            n0 = en0_v[pl.ds(o, 16)]
            n1 = en1_v[pl.ds(o, 16)]
            rij = rij_v[pl.ds(o, 16)]
            lr0 = plsc.load_gather(lr0_v, [n0 * 4 + n1])
            lrr = lr0 - _ln16(rij)
            fc = 1.0 / (1.0 + jnp.exp((rij - _RS) * _WINV))
            for k in range(R):
                pk = pkc[k].at[et].get(mode=gsm)
                a1 = lax.bitcast_convert_type(pk & jnp.int32(-65536), jnp.float32)
                e2m1 = lax.bitcast_convert_type(pk << 16, jnp.float32)
                y = a1 * fc * jnp.exp(lrr + e2m1 * lrr)
                out_v[k, pl.ds(o, 16)] = y

        plsc.parallel_loop(0, G, 1, unroll=2)(group)

    bufA = (etA, en0A, en1A, rijA)
    bufB = (etB, en0B, en1B, rijB)

    issue_in(0, *bufA, sinA)

    def big(c2, carry):
        c0 = c2 * 2
        c1 = c0 + 1
        wait_in(*bufA, sinA)
        issue_in(c1, *bufB, sinB)

        @pl.when(c2 > 0)
        def _():
            wait_out(outA, soutA)

        compute(*bufA, outA)
        issue_out(c0, outA, soutA)

        wait_in(*bufB, sinB)

        @pl.when(c2 < NCH // 2 - 1)
        def _():
            issue_in(c0 + 2, *bufA, sinA)

        @pl.when(c2 > 0)
        def _():
            wait_out(outB, soutB)

        compute(*bufB, outB)
        issue_out(c1, outB, soutB)
        return carry

    lax.fori_loop(0, NCH // 2, big, 0)
    wait_out(outA, soutA)
    wait_out(outB, soutB)


_sk = functools.partial(
    pl.kernel,
    out_type=jax.ShapeDtypeStruct((R, E), jnp.float32),
    mesh=plsc.VectorSubcoreMesh(core_axis_name="c", subcore_axis_name="s"),
    compiler_params=pltpu.CompilerParams(
        needs_layout_passes=False, use_tc_tiling_on_sc=True),
    scratch_types=[
        pltpu.VMEM((2 * R * 16,), jnp.float32),
        pltpu.VMEM((16,), jnp.float32),
        pltpu.VMEM((C,), jnp.int32),
        pltpu.VMEM((C,), jnp.int32),
        pltpu.VMEM((C,), jnp.int32),
        pltpu.VMEM((C,), jnp.float32),
        pltpu.VMEM((R, C), jnp.float32),
        pltpu.VMEM((C,), jnp.int32),
        pltpu.VMEM((C,), jnp.int32),
        pltpu.VMEM((C,), jnp.int32),
        pltpu.VMEM((C,), jnp.float32),
        pltpu.VMEM((R, C), jnp.float32),
        pltpu.SemaphoreType.DMA,
        pltpu.SemaphoreType.DMA,
        pltpu.SemaphoreType.DMA,
        pltpu.SemaphoreType.DMA,
    ],
)(_body)


def kernel(edge_type, edge_number, rij, hopping_param, atomic_radius):
    tabT = jnp.transpose(hopping_param, (1, 2, 0)).reshape(2 * R * NT)
    r0p = ((atomic_radius[:, None] + atomic_radius[None, :]) / _BOHR).reshape(NT)
    en_flat = edge_number.reshape(2 * E)
    out = _sk(edge_type, en_flat, rij, tabT, r0p)
    return out.T


# speedup vs baseline: 1.1680x; 1.0239x over previous
import functools

import jax
import jax.numpy as jnp
from jax import lax
from jax.experimental import pallas as pl
from jax.experimental.pallas import tpu as pltpu
from jax.experimental.pallas import tpu_sc as plsc

E = 1_600_000
R = 19
NT = 16
NW = 32
NBLK = E // 128
CBLK = 14
C = CBLK * 128
NCH = 28
G = C // 16

_LN2 = 0.6931471805599453
_BOHR = 1.8897259886
_RS = 6.0
_WINV = 5.0


def _lnpoly(z):
    z2 = z * z
    return z * (2.0 + z2 * (0.6666666666 + z2 * (0.4 + z2 * 0.2857142857)))


def _ln16(x):
    bits = lax.bitcast_convert_type(x, jnp.int32)
    e = (bits >> 23) - 127
    m = lax.bitcast_convert_type((bits & 0x007FFFFF) | 0x3F800000, jnp.float32)
    big = m > 1.4142135623730951
    m = jnp.where(big, m * 0.5, m)
    ef = e.astype(jnp.float32) + jnp.where(big, 1.0, 0.0)
    z = (m - 1.0) / (m + 1.0)
    return ef * _LN2 + _lnpoly(z)


def _body(et_hbm, en_hbm, rij_hbm, tab_hbm, r0p_hbm, out_hbm,
          tab_v, lr0_v,
          etA, en0A, en1A, rijA, outA,
          etB, en0B, en1B, rijB, outB,
          sinA, sinB, soutA, soutB):
    cid = lax.axis_index("c")
    sid = lax.axis_index("s")
    wid = sid * 2 + cid

    pltpu.sync_copy(tab_hbm, tab_v)
    pltpu.sync_copy(r0p_hbm, lr0_v)

    lr0_v[...] = _ln16(lr0_v[...])

    pkc = []
    for k in range(R):
        a1 = tab_v[pl.ds((2 * k) * 16, 16)]
        a2 = tab_v[pl.ds((2 * k + 1) * 16, 16)]
        hi = (lax.bitcast_convert_type(a1, jnp.int32) + 0x8000) & jnp.int32(-65536)
        lo = ((lax.bitcast_convert_type(jnp.abs(a2), jnp.int32) + 0x8000) >> 16) & 0xFFFF
        pkc.append(hi | lo)

    nb0 = (wid * (NBLK // 4)) >> 3
    nb1 = ((wid + 1) * (NBLK // 4)) >> 3
    gsm = lax.GatherScatterMode.PROMISE_IN_BOUNDS

    def cb_of(c):
        return jnp.minimum(nb0 + c * CBLK, nb1 - CBLK) * 128

    def issue_in(c, et_v, en0_v, en1_v, rij_v, sem):
        cb = cb_of(c)
        pltpu.async_copy(et_hbm.at[pl.ds(cb, C)], et_v, sem)
        pltpu.async_copy(en_hbm.at[pl.ds(cb, C)], en0_v, sem)
        pltpu.async_copy(en_hbm.at[pl.ds(E + cb, C)], en1_v, sem)
        pltpu.async_copy(rij_hbm.at[pl.ds(cb, C)], rij_v, sem)

    def wait_in(et_v, en0_v, en1_v, rij_v, sem):
        pltpu.make_async_copy(et_hbm.at[pl.ds(0, C)], et_v, sem).wait()
        pltpu.make_async_copy(en_hbm.at[pl.ds(0, C)], en0_v, sem).wait()
        pltpu.make_async_copy(en_hbm.at[pl.ds(0, C)], en1_v, sem).wait()
        pltpu.make_async_copy(rij_hbm.at[pl.ds(0, C)], rij_v, sem).wait()

    def issue_out(c, out_v, sem):
        pltpu.async_copy(out_v, out_hbm.at[:, pl.ds(cb_of(c), C)], sem)

    def wait_out(out_v, sem):
        pltpu.make_async_copy(out_v, out_hbm.at[:, pl.ds(0, C)], sem).wait()

    def compute(et_v, en0_v, en1_v, rij_v, out_v):
        def group(g):
            o = pl.multiple_of(g * 16, 16)
            et = et_v[pl.ds(o, 16)]
            n0 = en0_v[pl.ds(o, 16)]
            n1 = en1_v[pl.ds(o, 16)]
            rij = rij_v[pl.ds(o, 16)]
            lr0 = plsc.load_gather(lr0_v, [n0 * 4 + n1])
            lrr = lr0 - _ln16(rij)
            fc = 1.0 / (1.0 + jnp.exp((rij - _RS) * _WINV))
            for k in range(R):
                pk = pkc[k].at[et].get(mode=gsm)
                a1 = lax.bitcast_convert_type(pk & jnp.int32(-65536), jnp.float32)
                e2m1 = lax.bitcast_convert_type(pk << 16, jnp.float32)
                y = a1 * fc * jnp.exp(lrr + e2m1 * lrr)
                out_v[k, pl.ds(o, 16)] = y

        plsc.parallel_loop(0, G, 1, unroll=4)(group)

    bufA = (etA, en0A, en1A, rijA)
    bufB = (etB, en0B, en1B, rijB)

    issue_in(0, *bufA, sinA)

    def big(c2, carry):
        c0 = c2 * 2
        c1 = c0 + 1
        wait_in(*bufA, sinA)
        issue_in(c1, *bufB, sinB)

        @pl.when(c2 > 0)
        def _():
            wait_out(outA, soutA)

        compute(*bufA, outA)
        issue_out(c0, outA, soutA)

        wait_in(*bufB, sinB)

        @pl.when(c2 < NCH // 2 - 1)
        def _():
            issue_in(c0 + 2, *bufA, sinA)

        @pl.when(c2 > 0)
        def _():
            wait_out(outB, soutB)

        compute(*bufB, outB)
        issue_out(c1, outB, soutB)
        return carry

    lax.fori_loop(0, NCH // 2, big, 0)
    wait_out(outA, soutA)
    wait_out(outB, soutB)


_sk = functools.partial(
    pl.kernel,
    out_type=jax.ShapeDtypeStruct((R, E), jnp.float32),
    mesh=plsc.VectorSubcoreMesh(core_axis_name="c", subcore_axis_name="s"),
    compiler_params=pltpu.CompilerParams(
        needs_layout_passes=False, use_tc_tiling_on_sc=True),
    scratch_types=[
        pltpu.VMEM((2 * R * 16,), jnp.float32),
        pltpu.VMEM((16,), jnp.float32),
        pltpu.VMEM((C,), jnp.int32),
        pltpu.VMEM((C,), jnp.int32),
        pltpu.VMEM((C,), jnp.int32),
        pltpu.VMEM((C,), jnp.float32),
        pltpu.VMEM((R, C), jnp.float32),
        pltpu.VMEM((C,), jnp.int32),
        pltpu.VMEM((C,), jnp.int32),
        pltpu.VMEM((C,), jnp.int32),
        pltpu.VMEM((C,), jnp.float32),
        pltpu.VMEM((R, C), jnp.float32),
        pltpu.SemaphoreType.DMA,
        pltpu.SemaphoreType.DMA,
        pltpu.SemaphoreType.DMA,
        pltpu.SemaphoreType.DMA,
    ],
)(_body)


def kernel(edge_type, edge_number, rij, hopping_param, atomic_radius):
    tabT = jnp.transpose(hopping_param, (1, 2, 0)).reshape(2 * R * NT)
    r0p = ((atomic_radius[:, None] + atomic_radius[None, :]) / _BOHR).reshape(NT)
    en_flat = edge_number.reshape(2 * E)
    out = _sk(edge_type, en_flat, rij, tabT, r0p)
    return out.T


# speedup vs baseline: 1.3441x; 1.1508x over previous
import functools

import jax
import jax.numpy as jnp
from jax import lax
from jax.experimental import pallas as pl
from jax.experimental.pallas import tpu as pltpu
from jax.experimental.pallas import tpu_sc as plsc

E = 1_600_000
R = 19
NT = 16
NW = 32
NBLK = E // 128
CBLK = 14
C = CBLK * 128
NCH = 28
G = C // 16

_LN2 = 0.6931471805599453
_BOHR = 1.8897259886
_RS = 6.0
_WINV = 5.0


def _lnpoly(z):
    z2 = z * z
    return z * (2.0 + z2 * (0.6666666666 + z2 * (0.4 + z2 * 0.2857142857)))


def _ln16(x):
    bits = lax.bitcast_convert_type(x, jnp.int32)
    e = (bits >> 23) - 127
    m = lax.bitcast_convert_type((bits & 0x007FFFFF) | 0x3F800000, jnp.float32)
    big = m > 1.4142135623730951
    m = jnp.where(big, m * 0.5, m)
    ef = e.astype(jnp.float32) + jnp.where(big, 1.0, 0.0)
    z = (m - 1.0) / (m + 1.0)
    return ef * _LN2 + _lnpoly(z)


def _body(et_hbm, en_hbm, rij_hbm, tab_hbm, r0p_hbm, out_hbm,
          tab_v, lr0_v,
          etA, en0A, en1A, rijA, outA,
          etB, en0B, en1B, rijB, outB,
          sinA, sinB, soutA, soutB):
    cid = lax.axis_index("c")
    sid = lax.axis_index("s")
    wid = sid * 2 + cid

    pltpu.sync_copy(tab_hbm, tab_v)
    pltpu.sync_copy(r0p_hbm, lr0_v)

    lr0_v[...] = _ln16(lr0_v[...])

    pkc = []
    for k in range(R):
        a1 = tab_v[pl.ds((2 * k) * 16, 16)]
        a2 = tab_v[pl.ds((2 * k + 1) * 16, 16)]
        hi = (lax.bitcast_convert_type(a1, jnp.int32) + 0x8000) & jnp.int32(-65536)
        lo = ((lax.bitcast_convert_type(jnp.abs(a2), jnp.int32) + 0x8000) >> 16) & 0xFFFF
        pkc.append(hi | lo)

    nb0 = (wid * (NBLK // 4)) >> 3
    nb1 = ((wid + 1) * (NBLK // 4)) >> 3
    gsm = lax.GatherScatterMode.PROMISE_IN_BOUNDS

    def cb_of(c):
        return jnp.minimum(nb0 + c * CBLK, nb1 - CBLK) * 128

    def issue_in(c, et_v, en0_v, en1_v, rij_v, sem):
        cb = cb_of(c)
        pltpu.async_copy(et_hbm.at[pl.ds(cb, C)], et_v, sem)
        pltpu.async_copy(en_hbm.at[0, pl.ds(cb, C)], en0_v, sem)
        pltpu.async_copy(en_hbm.at[1, pl.ds(cb, C)], en1_v, sem)
        pltpu.async_copy(rij_hbm.at[pl.ds(cb, C)], rij_v, sem)

    def wait_in(et_v, en0_v, en1_v, rij_v, sem):
        pltpu.make_async_copy(et_hbm.at[pl.ds(0, C)], et_v, sem).wait()
        pltpu.make_async_copy(en_hbm.at[0, pl.ds(0, C)], en0_v, sem).wait()
        pltpu.make_async_copy(en_hbm.at[1, pl.ds(0, C)], en1_v, sem).wait()
        pltpu.make_async_copy(rij_hbm.at[pl.ds(0, C)], rij_v, sem).wait()

    def issue_out(c, out_v, sem):
        pltpu.async_copy(out_v, out_hbm.at[:, pl.ds(cb_of(c), C)], sem)

    def wait_out(out_v, sem):
        pltpu.make_async_copy(out_v, out_hbm.at[:, pl.ds(0, C)], sem).wait()

    def compute(et_v, en0_v, en1_v, rij_v, out_v):
        def group(g):
            o = pl.multiple_of(g * 16, 16)
            et = et_v[pl.ds(o, 16)]
            n0 = en0_v[pl.ds(o, 16)]
            n1 = en1_v[pl.ds(o, 16)]
            rij = rij_v[pl.ds(o, 16)]
            lr0 = plsc.load_gather(lr0_v, [n0 * 4 + n1])
            lrr = lr0 - _ln16(rij)
            fc = 1.0 / (1.0 + jnp.exp((rij - _RS) * _WINV))
            for k in range(R):
                pk = pkc[k].at[et].get(mode=gsm)
                a1 = lax.bitcast_convert_type(pk & jnp.int32(-65536), jnp.float32)
                e2m1 = lax.bitcast_convert_type(pk << 16, jnp.float32)
                y = a1 * fc * jnp.exp(lrr + e2m1 * lrr)
                out_v[k, pl.ds(o, 16)] = y

        plsc.parallel_loop(0, G, 1, unroll=4)(group)

    bufA = (etA, en0A, en1A, rijA)
    bufB = (etB, en0B, en1B, rijB)

    issue_in(0, *bufA, sinA)

    def big(c2, carry):
        c0 = c2 * 2
        c1 = c0 + 1
        wait_in(*bufA, sinA)
        issue_in(c1, *bufB, sinB)

        @pl.when(c2 > 0)
        def _():
            wait_out(outA, soutA)

        compute(*bufA, outA)
        issue_out(c0, outA, soutA)

        wait_in(*bufB, sinB)

        @pl.when(c2 < NCH // 2 - 1)
        def _():
            issue_in(c0 + 2, *bufA, sinA)

        @pl.when(c2 > 0)
        def _():
            wait_out(outB, soutB)

        compute(*bufB, outB)
        issue_out(c1, outB, soutB)
        return carry

    lax.fori_loop(0, NCH // 2, big, 0)
    wait_out(outA, soutA)
    wait_out(outB, soutB)


_sk = functools.partial(
    pl.kernel,
    out_type=jax.ShapeDtypeStruct((R, E), jnp.float32),
    mesh=plsc.VectorSubcoreMesh(core_axis_name="c", subcore_axis_name="s"),
    compiler_params=pltpu.CompilerParams(
        needs_layout_passes=False, use_tc_tiling_on_sc=True),
    scratch_types=[
        pltpu.VMEM((2 * R * 16,), jnp.float32),
        pltpu.VMEM((16,), jnp.float32),
        pltpu.VMEM((C,), jnp.int32),
        pltpu.VMEM((C,), jnp.int32),
        pltpu.VMEM((C,), jnp.int32),
        pltpu.VMEM((C,), jnp.float32),
        pltpu.VMEM((R, C), jnp.float32),
        pltpu.VMEM((C,), jnp.int32),
        pltpu.VMEM((C,), jnp.int32),
        pltpu.VMEM((C,), jnp.int32),
        pltpu.VMEM((C,), jnp.float32),
        pltpu.VMEM((R, C), jnp.float32),
        pltpu.SemaphoreType.DMA,
        pltpu.SemaphoreType.DMA,
        pltpu.SemaphoreType.DMA,
        pltpu.SemaphoreType.DMA,
    ],
)(_body)


def kernel(edge_type, edge_number, rij, hopping_param, atomic_radius):
    tabT = jnp.transpose(hopping_param, (1, 2, 0)).reshape(2 * R * NT)
    r0p = ((atomic_radius[:, None] + atomic_radius[None, :]) / _BOHR).reshape(NT)
    out = _sk(edge_type, edge_number, rij, tabT, r0p)
    return out.T
